# Initial kernel scaffold; baseline (speedup 1.0000x reference)
import jax
import jax.numpy as jnp
from jax.experimental import pallas as pl


def kernel(node, edge_index, edge_attr, batch_ptr, W_bases, W_comb, b_comb, b_conv, gn_weight, gn_bias, gn_mean_scale):
    raise NotImplementedError("write your pallas kernel here")


# speedup vs baseline: 2.3894x; 2.3894x over previous
import functools
import jax
import jax.numpy as jnp
from jax.experimental import pallas as pl
from jax.experimental.pallas import tpu as pltpu

HIDDEN = 128
HEADS = 8
BASES = 4
AGGRS = 3
F_HEAD = HIDDEN // HEADS
N_GRAPHS = 128
EPS = 1e-5

NBLK = 1000


def _d1_body(node_ref, wb_ref, wc_ref, bc_ref, deg_ref,
             bases_ref, basesd_ref, w_ref, dinv_ref):
    node = node_ref[...]
    bases = jnp.dot(node, wb_ref[...].T, preferred_element_type=jnp.float32)
    w = jnp.dot(node, wc_ref[...].T, preferred_element_type=jnp.float32)
    w_ref[...] = w + bc_ref[...][None, :]
    dinv = jax.lax.rsqrt(deg_ref[...])
    bases_ref[...] = bases
    basesd_ref[...] = bases * dinv
    dinv_ref[...] = dinv


def _d1(node, W_bases, W_comb, b_comb, deg):
    N = node.shape[0]
    g = N // NBLK
    out_shapes = (
        jax.ShapeDtypeStruct((N, BASES * F_HEAD), jnp.float32),
        jax.ShapeDtypeStruct((N, BASES * F_HEAD), jnp.float32),
        jax.ShapeDtypeStruct((N, HEADS * BASES * AGGRS), jnp.float32),
        jax.ShapeDtypeStruct((N, 1), jnp.float32),
    )
    return pl.pallas_call(
        _d1_body,
        grid=(g,),
        in_specs=[
            pl.BlockSpec((NBLK, HIDDEN), lambda i: (i, 0)),
            pl.BlockSpec((BASES * F_HEAD, HIDDEN), lambda i: (0, 0)),
            pl.BlockSpec((HEADS * BASES * AGGRS, HIDDEN), lambda i: (0, 0)),
            pl.BlockSpec((HEADS * BASES * AGGRS,), lambda i: (0,)),
            pl.BlockSpec((NBLK, 1), lambda i: (i, 0)),
        ],
        out_specs=(
            pl.BlockSpec((NBLK, BASES * F_HEAD), lambda i: (i, 0)),
            pl.BlockSpec((NBLK, BASES * F_HEAD), lambda i: (i, 0)),
            pl.BlockSpec((NBLK, HEADS * BASES * AGGRS), lambda i: (i, 0)),
            pl.BlockSpec((NBLK, 1), lambda i: (i, 0)),
        ),
        out_shape=out_shapes,
    )(node, W_bases, W_comb, b_comb, deg)


def _onehot(batch):
    n = batch.shape[0]
    gid = jax.lax.broadcasted_iota(jnp.int32, (n, N_GRAPHS), 1)
    return (batch == gid).astype(jnp.float32)


def _combine(agg_sym, agg_sum, agg_max, w):
    n = w.shape[0]
    aggs = (agg_sym, agg_sum, agg_max)
    h_cols = []
    for h in range(HEADS):
        acc = jnp.zeros((n, F_HEAD), jnp.float32)
        for k in range(BASES * AGGRS):
            a = aggs[k // BASES][:, (k % BASES) * F_HEAD:(k % BASES + 1) * F_HEAD]
            acc = acc + w[:, h * BASES * AGGRS + k][:, None] * a
        h_cols.append(acc)
    return jnp.concatenate(h_cols, axis=1)


def _d2a_body(aggsym_ref, aggsum_ref, aggmax_ref, bases_ref, dinv_ref,
              w_ref, bconv_ref, batch_ref,
              hmat_ref, sums_ref, cnt_ref):
    dinv = dinv_ref[...]
    bases = bases_ref[...]
    agg_sym = aggsym_ref[...] * dinv + bases * (dinv * dinv)
    agg_sum = aggsum_ref[...] + bases
    agg_max = jnp.maximum(aggmax_ref[...], bases)
    hmat = _combine(agg_sym, agg_sum, agg_max, w_ref[...])
    hmat = hmat + bconv_ref[...][None, :]
    hmat_ref[...] = hmat
    onehot = _onehot(batch_ref[...])
    ohT = onehot.T
    psum = jnp.dot(ohT, hmat, preferred_element_type=jnp.float32)
    pcnt = jnp.dot(ohT, jnp.ones_like(hmat), preferred_element_type=jnp.float32)

    @pl.when(pl.program_id(0) == 0)
    def _init():
        sums_ref[...] = jnp.zeros_like(sums_ref)
        cnt_ref[...] = jnp.zeros_like(cnt_ref)

    sums_ref[...] += psum
    cnt_ref[...] += pcnt


def _d2a(agg_sym, agg_sum, agg_max, bases, dinv, w, b_conv, batch_ptr):
    N = w.shape[0]
    g = N // NBLK
    F64 = BASES * F_HEAD
    return pl.pallas_call(
        _d2a_body,
        grid=(g,),
        in_specs=[
            pl.BlockSpec((NBLK, F64), lambda i: (i, 0)),
            pl.BlockSpec((NBLK, F64), lambda i: (i, 0)),
            pl.BlockSpec((NBLK, F64), lambda i: (i, 0)),
            pl.BlockSpec((NBLK, F64), lambda i: (i, 0)),
            pl.BlockSpec((NBLK, 1), lambda i: (i, 0)),
            pl.BlockSpec((NBLK, HEADS * BASES * AGGRS), lambda i: (i, 0)),
            pl.BlockSpec((HIDDEN,), lambda i: (0,)),
            pl.BlockSpec((NBLK, 1), lambda i: (i, 0)),
        ],
        out_specs=(
            pl.BlockSpec((NBLK, HIDDEN), lambda i: (i, 0)),
            pl.BlockSpec((N_GRAPHS, HIDDEN), lambda i: (0, 0)),
            pl.BlockSpec((N_GRAPHS, HIDDEN), lambda i: (0, 0)),
        ),
        out_shape=(
            jax.ShapeDtypeStruct((N, HIDDEN), jnp.float32),
            jax.ShapeDtypeStruct((N_GRAPHS, HIDDEN), jnp.float32),
            jax.ShapeDtypeStruct((N_GRAPHS, HIDDEN), jnp.float32),
        ),
    )(agg_sym, agg_sum, agg_max, bases, dinv, w, b_conv, batch_ptr)


def _d2b_body(hmat_ref, batch_ref, sums_ref, cnt_ref, gnm_ref,
              out_ref, var_ref):
    cnt = jnp.maximum(cnt_ref[...], 1.0)
    mean = sums_ref[...] / cnt
    onehot = _onehot(batch_ref[...])
    mean_n = jnp.dot(onehot, mean, preferred_element_type=jnp.float32)
    out = hmat_ref[...] - mean_n * gnm_ref[...][None, :]
    out_ref[...] = out
    pvar = jnp.dot(onehot.T, out * out, preferred_element_type=jnp.float32)

    @pl.when(pl.program_id(0) == 0)
    def _init():
        var_ref[...] = jnp.zeros_like(var_ref)

    var_ref[...] += pvar


def _d2b(hmat, batch_ptr, sums, cnt, gn_mean_scale):
    N = hmat.shape[0]
    g = N // NBLK
    return pl.pallas_call(
        _d2b_body,
        grid=(g,),
        in_specs=[
            pl.BlockSpec((NBLK, HIDDEN), lambda i: (i, 0)),
            pl.BlockSpec((NBLK, 1), lambda i: (i, 0)),
            pl.BlockSpec((N_GRAPHS, HIDDEN), lambda i: (0, 0)),
            pl.BlockSpec((N_GRAPHS, HIDDEN), lambda i: (0, 0)),
            pl.BlockSpec((HIDDEN,), lambda i: (0,)),
        ],
        out_specs=(
            pl.BlockSpec((NBLK, HIDDEN), lambda i: (i, 0)),
            pl.BlockSpec((N_GRAPHS, HIDDEN), lambda i: (0, 0)),
        ),
        out_shape=(
            jax.ShapeDtypeStruct((N, HIDDEN), jnp.float32),
            jax.ShapeDtypeStruct((N_GRAPHS, HIDDEN), jnp.float32),
        ),
    )(hmat, batch_ptr, sums, cnt, gn_mean_scale)


def _d2c_body(out_ref, batch_ref, var_ref, cnt_ref, gnw_ref, gnb_ref,
              res_ref):
    cnt = jnp.maximum(cnt_ref[...], 1.0)
    std = jnp.sqrt(var_ref[...] / cnt + EPS)
    onehot = _onehot(batch_ref[...])
    std_n = jnp.dot(onehot, std, preferred_element_type=jnp.float32)
    res = gnw_ref[...][None, :] * out_ref[...] / std_n + gnb_ref[...][None, :]
    res_ref[...] = jnp.maximum(res, 0.0)


def _d2c(out, batch_ptr, var, cnt, gn_weight, gn_bias):
    N = out.shape[0]
    g = N // NBLK
    return pl.pallas_call(
        _d2c_body,
        grid=(g,),
        in_specs=[
            pl.BlockSpec((NBLK, HIDDEN), lambda i: (i, 0)),
            pl.BlockSpec((NBLK, 1), lambda i: (i, 0)),
            pl.BlockSpec((N_GRAPHS, HIDDEN), lambda i: (0, 0)),
            pl.BlockSpec((N_GRAPHS, HIDDEN), lambda i: (0, 0)),
            pl.BlockSpec((HIDDEN,), lambda i: (0,)),
            pl.BlockSpec((HIDDEN,), lambda i: (0,)),
        ],
        out_specs=pl.BlockSpec((NBLK, HIDDEN), lambda i: (i, 0)),
        out_shape=jax.ShapeDtypeStruct((N, HIDDEN), jnp.float32),
    )(out, batch_ptr, var, cnt, gn_weight, gn_bias)


def kernel(node, edge_index, edge_attr, batch_ptr, W_bases, W_comb, b_comb,
           b_conv, gn_weight, gn_bias, gn_mean_scale):
    N = node.shape[0]
    E = edge_index.shape[1]
    row, col = edge_index[0], edge_index[1]

    deg = jax.ops.segment_sum(jnp.ones((E,), jnp.float32), col,
                              num_segments=N) + 1.0

    bases, basesd, w, dinv = _d1(node, W_bases, W_comb, b_comb, deg[:, None])

    agg_sym = jax.ops.segment_sum(basesd[row], col, num_segments=N)
    agg_sum = jax.ops.segment_sum(bases[row], col, num_segments=N)
    agg_max = jax.ops.segment_max(bases[row], col, num_segments=N)
    agg_max = jnp.maximum(agg_max, jnp.float32(-3.0e38))

    batch2 = batch_ptr[:, None]
    hmat, sums, cnt = _d2a(agg_sym, agg_sum, agg_max, bases, dinv, w,
                           b_conv, batch2)
    out, var = _d2b(hmat, batch2, sums, cnt, gn_mean_scale)
    return _d2c(out, batch2, var, cnt, gn_weight, gn_bias)


# speedup vs baseline: 3.7125x; 1.5537x over previous
import functools
import jax
import jax.numpy as jnp
from jax import lax
from jax.experimental import pallas as pl
from jax.experimental.pallas import tpu as pltpu
from jax.experimental.pallas import tpu_sc as plsc

HIDDEN = 128
HEADS = 8
BASES = 4
AGGRS = 3
F_HEAD = HIDDEN // HEADS
N_GRAPHS = 128
EPS = 1e-5

NBLK = 1000

NCORE = 2
NSUB = 16
NTILE = NCORE * NSUB
EPAD = 327680
EROWS = EPAD // 128
TROWS = EROWS // NTILE
NPAD = 10240
NSLICE = NPAD // NSUB
DEGW = 128


def _d1a_body(node_ref, wb_ref, wc_ref, bc_ref, bases_ref, w_ref):
    node = node_ref[...]
    bases = jnp.dot(node, wb_ref[...].T, preferred_element_type=jnp.float32)
    w = jnp.dot(node, wc_ref[...].T, preferred_element_type=jnp.float32)
    w_ref[...] = w + bc_ref[...][None, :]
    bases_ref[...] = bases


def _d1a(node, W_bases, W_comb, b_comb):
    N = node.shape[0]
    g = N // NBLK
    return pl.pallas_call(
        _d1a_body,
        grid=(g,),
        in_specs=[
            pl.BlockSpec((NBLK, HIDDEN), lambda i: (i, 0)),
            pl.BlockSpec((BASES * F_HEAD, HIDDEN), lambda i: (0, 0)),
            pl.BlockSpec((HEADS * BASES * AGGRS, HIDDEN), lambda i: (0, 0)),
            pl.BlockSpec((HEADS * BASES * AGGRS,), lambda i: (0,)),
        ],
        out_specs=(
            pl.BlockSpec((NBLK, BASES * F_HEAD), lambda i: (i, 0)),
            pl.BlockSpec((NBLK, HEADS * BASES * AGGRS), lambda i: (i, 0)),
        ),
        out_shape=(
            jax.ShapeDtypeStruct((N, BASES * F_HEAD), jnp.float32),
            jax.ShapeDtypeStruct((N, HEADS * BASES * AGGRS), jnp.float32),
        ),
    )(node, W_bases, W_comb, b_comb)


def _d1b_body(bases_ref, dega_ref, degb_ref, comb_ref, dinv_ref):
    deg = dega_ref[...][:, 0:1] + degb_ref[...][:, 0:1] + 1.0
    dinv = jax.lax.rsqrt(deg)
    bases = bases_ref[...]
    comb_ref[...] = jnp.concatenate([bases, bases * dinv], axis=1)
    dinv_ref[...] = dinv


def _d1b(bases, deg_a, deg_b):
    N = bases.shape[0]
    g = N // NBLK
    return pl.pallas_call(
        _d1b_body,
        grid=(g,),
        in_specs=[
            pl.BlockSpec((NBLK, BASES * F_HEAD), lambda i: (i, 0)),
            pl.BlockSpec((NBLK, DEGW), lambda i: (i, 0)),
            pl.BlockSpec((NBLK, DEGW), lambda i: (i, 0)),
        ],
        out_specs=(
            pl.BlockSpec((NBLK, HIDDEN), lambda i: (i, 0)),
            pl.BlockSpec((NBLK, 1), lambda i: (i, 0)),
        ),
        out_shape=(
            jax.ShapeDtypeStruct((N, HIDDEN), jnp.float32),
            jax.ShapeDtypeStruct((N, 1), jnp.float32),
        ),
    )(bases, deg_a, deg_b)


def _sc_deg_kernel(col_hbm, out_hbm, idxc, ones, acc):
    c = lax.axis_index("c")
    s = lax.axis_index("s")
    wid = c * NSUB + s

    @pl.loop(0, 128)
    def _(i):
        @pl.loop(0, DEGW, step=16)
        def _(q):
            ones[i, pl.ds(q, 16)] = jnp.zeros((16,), jnp.float32)

    base = s * NSLICE
    @pl.loop(0, NSLICE, step=128)
    def _(off):
        pltpu.sync_copy(ones.at[pl.ds(0, 128)], acc.at[pl.ds(base + off, 128)])

    @pl.loop(0, 128)
    def _(i):
        @pl.loop(0, DEGW, step=16)
        def _(q):
            ones[i, pl.ds(q, 16)] = jnp.ones((16,), jnp.float32)

    plsc.subcore_barrier()

    pltpu.sync_copy(col_hbm.at[pl.ds(wid * TROWS, TROWS)], idxc)

    @pl.loop(0, TROWS)
    def _(j):
        pltpu.sync_copy(ones, acc.at[idxc.at[j]], add=True)

    plsc.subcore_barrier()
    pltpu.sync_copy(acc.at[pl.ds(base, NSLICE)],
                    out_hbm.at[c].at[pl.ds(base, NSLICE)])


def _sc_deg(col2):
    mesh = plsc.VectorSubcoreMesh(core_axis_name="c", subcore_axis_name="s")
    k = pl.kernel(
        _sc_deg_kernel,
        out_type=jax.ShapeDtypeStruct((NCORE, NPAD, DEGW), jnp.float32),
        mesh=mesh,
        scratch_types=[
            pltpu.VMEM((TROWS, 128), jnp.int32),
            pltpu.VMEM((128, DEGW), jnp.float32),
            pltpu.VMEM_SHARED((NPAD, DEGW), jnp.float32),
        ],
    )
    return k(col2)


def _sc_sums_kernel(comb_hbm, row_hbm, col_hbm, out, idxr, idxc, gb, acc):
    c = lax.axis_index("c")
    s = lax.axis_index("s")
    wid = c * NSUB + s

    @pl.loop(0, 128)
    def _(i):
        @pl.loop(0, HIDDEN, step=16)
        def _(q):
            gb[i, pl.ds(q, 16)] = jnp.zeros((16,), jnp.float32)

    base = s * NSLICE

    @pl.loop(0, NSLICE, step=128)
    def _(off):
        pltpu.sync_copy(gb, acc.at[pl.ds(base + off, 128)])

    plsc.subcore_barrier()

    pltpu.sync_copy(row_hbm.at[pl.ds(wid * TROWS, TROWS)], idxr)
    pltpu.sync_copy(col_hbm.at[pl.ds(wid * TROWS, TROWS)], idxc)

    @pl.loop(0, TROWS)
    def _(j):
        pltpu.sync_copy(comb_hbm.at[idxr.at[j]], gb)
        pltpu.sync_copy(gb, acc.at[idxc.at[j]], add=True)

    plsc.subcore_barrier()
    pltpu.sync_copy(acc.at[pl.ds(base, NSLICE)],
                    out.at[c].at[pl.ds(base, NSLICE)])


def _sc_sums(comb, row2, col2):
    mesh = plsc.VectorSubcoreMesh(core_axis_name="c", subcore_axis_name="s")
    k = pl.kernel(
        _sc_sums_kernel,
        out_type=jax.ShapeDtypeStruct((NCORE, NPAD, HIDDEN), jnp.float32),
        mesh=mesh,
        scratch_types=[
            pltpu.VMEM((TROWS, 128), jnp.int32),
            pltpu.VMEM((TROWS, 128), jnp.int32),
            pltpu.VMEM((128, HIDDEN), jnp.float32),
            pltpu.VMEM_SHARED((NPAD, HIDDEN), jnp.float32),
        ],
    )
    return k(comb, row2, col2)


def _onehot(batch):
    n = batch.shape[0]
    gid = jax.lax.broadcasted_iota(jnp.int32, (n, N_GRAPHS), 1)
    return (batch == gid).astype(jnp.float32)


def _combine(agg_sym, agg_sum, agg_max, w):
    n = w.shape[0]
    aggs = (agg_sym, agg_sum, agg_max)
    h_cols = []
    for h in range(HEADS):
        acc = jnp.zeros((n, F_HEAD), jnp.float32)
        for k in range(BASES * AGGRS):
            a = aggs[k // BASES][:, (k % BASES) * F_HEAD:(k % BASES + 1) * F_HEAD]
            acc = acc + w[:, h * BASES * AGGRS + k][:, None] * a
        h_cols.append(acc)
    return jnp.concatenate(h_cols, axis=1)


def _d2a_body(p0_ref, p1_ref, aggmax_ref,
              bases_ref, dinv_ref, w_ref, bconv_ref, batch_ref,
              hmat_ref, sums_ref, cnt_ref):
    dinv = dinv_ref[...]
    bases = bases_ref[...]
    F64 = BASES * F_HEAD
    p = p0_ref[...] + p1_ref[...]
    agg_sym = p[:, F64:] * dinv + bases * (dinv * dinv)
    agg_sum = p[:, :F64] + bases
    agg_max = jnp.maximum(aggmax_ref[...], bases)
    hmat = _combine(agg_sym, agg_sum, agg_max, w_ref[...])
    hmat = hmat + bconv_ref[...][None, :]
    hmat_ref[...] = hmat
    onehot = _onehot(batch_ref[...])
    ohT = onehot.T
    psum = jnp.dot(ohT, hmat, preferred_element_type=jnp.float32)
    pcnt = jnp.dot(ohT, jnp.ones_like(hmat), preferred_element_type=jnp.float32)

    @pl.when(pl.program_id(0) == 0)
    def _init():
        sums_ref[...] = jnp.zeros_like(sums_ref)
        cnt_ref[...] = jnp.zeros_like(cnt_ref)

    sums_ref[...] += psum
    cnt_ref[...] += pcnt


def _d2a(part0, part1, agg_max, bases, dinv, w, b_conv, batch_ptr):
    N = w.shape[0]
    g = N // NBLK
    F64 = BASES * F_HEAD
    return pl.pallas_call(
        _d2a_body,
        grid=(g,),
        in_specs=[
            pl.BlockSpec((NBLK, HIDDEN), lambda i: (i, 0)),
            pl.BlockSpec((NBLK, HIDDEN), lambda i: (i, 0)),
            pl.BlockSpec((NBLK, F64), lambda i: (i, 0)),
            pl.BlockSpec((NBLK, F64), lambda i: (i, 0)),
            pl.BlockSpec((NBLK, 1), lambda i: (i, 0)),
            pl.BlockSpec((NBLK, HEADS * BASES * AGGRS), lambda i: (i, 0)),
            pl.BlockSpec((HIDDEN,), lambda i: (0,)),
            pl.BlockSpec((NBLK, 1), lambda i: (i, 0)),
        ],
        out_specs=(
            pl.BlockSpec((NBLK, HIDDEN), lambda i: (i, 0)),
            pl.BlockSpec((N_GRAPHS, HIDDEN), lambda i: (0, 0)),
            pl.BlockSpec((N_GRAPHS, HIDDEN), lambda i: (0, 0)),
        ),
        out_shape=(
            jax.ShapeDtypeStruct((N, HIDDEN), jnp.float32),
            jax.ShapeDtypeStruct((N_GRAPHS, HIDDEN), jnp.float32),
            jax.ShapeDtypeStruct((N_GRAPHS, HIDDEN), jnp.float32),
        ),
    )(part0, part1, agg_max, bases, dinv, w, b_conv, batch_ptr)


def _d2b_body(hmat_ref, batch_ref, sums_ref, cnt_ref, gnm_ref,
              out_ref, var_ref):
    cnt = jnp.maximum(cnt_ref[...], 1.0)
    mean = sums_ref[...] / cnt
    onehot = _onehot(batch_ref[...])
    mean_n = jnp.dot(onehot, mean, preferred_element_type=jnp.float32)
    out = hmat_ref[...] - mean_n * gnm_ref[...][None, :]
    out_ref[...] = out
    pvar = jnp.dot(onehot.T, out * out, preferred_element_type=jnp.float32)

    @pl.when(pl.program_id(0) == 0)
    def _init():
        var_ref[...] = jnp.zeros_like(var_ref)

    var_ref[...] += pvar


def _d2b(hmat, batch_ptr, sums, cnt, gn_mean_scale):
    N = hmat.shape[0]
    g = N // NBLK
    return pl.pallas_call(
        _d2b_body,
        grid=(g,),
        in_specs=[
            pl.BlockSpec((NBLK, HIDDEN), lambda i: (i, 0)),
            pl.BlockSpec((NBLK, 1), lambda i: (i, 0)),
            pl.BlockSpec((N_GRAPHS, HIDDEN), lambda i: (0, 0)),
            pl.BlockSpec((N_GRAPHS, HIDDEN), lambda i: (0, 0)),
            pl.BlockSpec((HIDDEN,), lambda i: (0,)),
        ],
        out_specs=(
            pl.BlockSpec((NBLK, HIDDEN), lambda i: (i, 0)),
            pl.BlockSpec((N_GRAPHS, HIDDEN), lambda i: (0, 0)),
        ),
        out_shape=(
            jax.ShapeDtypeStruct((N, HIDDEN), jnp.float32),
            jax.ShapeDtypeStruct((N_GRAPHS, HIDDEN), jnp.float32),
        ),
    )(hmat, batch_ptr, sums, cnt, gn_mean_scale)


def _d2c_body(out_ref, batch_ref, var_ref, cnt_ref, gnw_ref, gnb_ref,
              res_ref):
    cnt = jnp.maximum(cnt_ref[...], 1.0)
    std = jnp.sqrt(var_ref[...] / cnt + EPS)
    onehot = _onehot(batch_ref[...])
    std_n = jnp.dot(onehot, std, preferred_element_type=jnp.float32)
    res = gnw_ref[...][None, :] * out_ref[...] / std_n + gnb_ref[...][None, :]
    res_ref[...] = jnp.maximum(res, 0.0)


def _d2c(out, batch_ptr, var, cnt, gn_weight, gn_bias):
    N = out.shape[0]
    g = N // NBLK
    return pl.pallas_call(
        _d2c_body,
        grid=(g,),
        in_specs=[
            pl.BlockSpec((NBLK, HIDDEN), lambda i: (i, 0)),
            pl.BlockSpec((NBLK, 1), lambda i: (i, 0)),
            pl.BlockSpec((N_GRAPHS, HIDDEN), lambda i: (0, 0)),
            pl.BlockSpec((N_GRAPHS, HIDDEN), lambda i: (0, 0)),
            pl.BlockSpec((HIDDEN,), lambda i: (0,)),
            pl.BlockSpec((HIDDEN,), lambda i: (0,)),
        ],
        out_specs=pl.BlockSpec((NBLK, HIDDEN), lambda i: (i, 0)),
        out_shape=jax.ShapeDtypeStruct((N, HIDDEN), jnp.float32),
    )(out, batch_ptr, var, cnt, gn_weight, gn_bias)


def kernel(node, edge_index, edge_attr, batch_ptr, W_bases, W_comb, b_comb,
           b_conv, gn_weight, gn_bias, gn_mean_scale):
    N = node.shape[0]
    E = edge_index.shape[1]
    row, col = edge_index[0], edge_index[1]

    npadded = EPAD - E
    pad_ids = jnp.arange(npadded, dtype=jnp.int32)
    rowp = jnp.concatenate([row, (pad_ids * 97) % N])
    colp = jnp.concatenate([col, N + 8 + (pad_ids % 8)])
    row2 = rowp.reshape(EROWS, 128)
    col2 = colp.reshape(EROWS, 128)

    deg2 = _sc_deg(col2)

    bases, w = _d1a(node, W_bases, W_comb, b_comb)
    comb, dinv = _d1b(bases, deg2[0, :N, :], deg2[1, :N, :])

    parts = _sc_sums(comb, row2, col2)

    agg_max = jax.ops.segment_max(bases[row], col, num_segments=N)
    agg_max = jnp.maximum(agg_max, jnp.float32(-3.0e38))

    batch2 = batch_ptr[:, None]
    hmat, sums, cnt = _d2a(parts[0, :N], parts[1, :N], agg_max, bases,
                           dinv, w, b_conv, batch2)
    out, var = _d2b(hmat, batch2, sums, cnt, gn_mean_scale)
    return _d2c(out, batch2, var, cnt, gn_weight, gn_bias)
